# Initial kernel scaffold; baseline (speedup 1.0000x reference)
import jax
import jax.numpy as jnp
from jax.experimental import pallas as pl


def kernel(hidden_states, W_gate, W1, W3, W2):
    raise NotImplementedError("write your pallas kernel here")


# speedup vs baseline: 1.0683x; 1.0683x over previous
import jax
import jax.numpy as jnp
from jax.experimental import pallas as pl
from jax.experimental.pallas import tpu as pltpu

BT = 1024
BF = 512


def _ffn_body(x_ref, w1_ref, w3_ref, w2_ref, o_ref):
    @pl.when(pl.program_id(1) == 0)
    def _init():
        o_ref[...] = jnp.zeros_like(o_ref)

    x = x_ref[...]
    a = jnp.dot(x, w1_ref[...], preferred_element_type=jnp.float32)
    b = jnp.dot(x, w3_ref[...], preferred_element_type=jnp.float32)
    h = a * b
    act = h * jax.nn.sigmoid(h)
    o_ref[...] += jnp.dot(act, w2_ref[...], preferred_element_type=jnp.float32)


def kernel(hidden_states, W_gate, W1, W3, W2):
    T, D = hidden_states.shape
    F = W1.shape[1]
    nt, nf = T // BT, F // BF
    return pl.pallas_call(
        _ffn_body,
        grid=(nt, nf),
        in_specs=[
            pl.BlockSpec((BT, D), lambda t, f: (t, 0)),
            pl.BlockSpec((D, BF), lambda t, f: (0, f)),
            pl.BlockSpec((D, BF), lambda t, f: (0, f)),
            pl.BlockSpec((BF, D), lambda t, f: (f, 0)),
        ],
        out_specs=pl.BlockSpec((BT, D), lambda t, f: (t, 0)),
        out_shape=jax.ShapeDtypeStruct((T, D), jnp.float32),
        compiler_params=pltpu.CompilerParams(
            dimension_semantics=("arbitrary", "arbitrary"),
        ),
    )(hidden_states, W1, W3, W2)
